# baseline (speedup 1.0000x reference)
import functools

import jax
import jax.numpy as jnp
from jax import lax
from jax.experimental import pallas as pl
from jax.experimental.pallas import tpu as pltpu

B = 16384
DIM = 128
VIS = 4096
BM = 256
NB = B // BM


def _score_block(bh_ref, bt_ref, tms_ref,
                 visual_ref,
                 hre_ref, him_ref, h_ref, tre_ref, tim_ref, t_ref,
                 rre_ref, rim_ref, r_ref, tm_ref, wp_ref, bp_ref,
                 out_ref,
                 xh_buf, xt_buf, sem):
    i = pl.program_id(0)
    nb = pl.num_programs(0)

    def issue(block, slot):
        def body(j, carry):
            row = block * BM + j

            @pl.when(tms_ref[row] != 0)
            def _():
                pltpu.make_async_copy(
                    visual_ref.at[bh_ref[row]], xh_buf.at[slot, j],
                    sem.at[slot, 0]).start()
                pltpu.make_async_copy(
                    visual_ref.at[bt_ref[row]], xt_buf.at[slot, j],
                    sem.at[slot, 1]).start()
            return carry
        lax.fori_loop(0, BM, body, 0, unroll=8)

    def wait(block, slot):
        def body(j, carry):
            row = block * BM + j

            @pl.when(tms_ref[row] != 0)
            def _():
                pltpu.make_async_copy(
                    visual_ref.at[bh_ref[row]], xh_buf.at[slot, j],
                    sem.at[slot, 0]).wait()
                pltpu.make_async_copy(
                    visual_ref.at[bt_ref[row]], xt_buf.at[slot, j],
                    sem.at[slot, 1]).wait()
            return carry
        lax.fori_loop(0, BM, body, 0, unroll=8)

    @pl.when(i == 0)
    def _():
        issue(0, 0)

    @pl.when(i + 1 < nb)
    def _():
        issue(i + 1, (i + 1) % 2)

    slot = i % 2
    wait(i, slot)

    xh = xh_buf[slot]
    xt = xt_buf[slot]
    wp = wp_ref[...]
    bp = bp_ref[...]
    ha = lax.dot_general(xh, wp, (((1,), (1,)), ((), ())),
                         preferred_element_type=jnp.float32) + bp
    ta = lax.dot_general(xt, wp, (((1,), (1,)), ((), ())),
                         preferred_element_type=jnp.float32) + bp
    hre = hre_ref[...]
    him = him_ref[...]
    tre = tre_ref[...]
    tim = tim_ref[...]
    rre = rre_ref[...]
    rim = rim_ref[...]
    h = h_ref[...]
    t = t_ref[...]
    r = r_ref[...]
    c = -jnp.sum(rre * (hre * tre + him * tim) + rim * (hre * tim - him * tre),
                 axis=-1)
    s_tt = jnp.sum(h * t * r, axis=-1)
    s_it = jnp.sum(ha * t * r, axis=-1)
    s_ti = jnp.sum(h * ta * r, axis=-1)
    s_ii = jnp.sum(ha * ta * r, axis=-1)
    tm = tm_ref[0, ...]
    score = jnp.where(tm == 0, c - s_tt, 0.0)
    score = score + jnp.where(tm == 1, 2.0 * c - s_it - s_ti, 0.0)
    score = score + jnp.where(tm == 2, c - s_ii, 0.0)
    out_ref[0, ...] = score


@jax.jit
def _fused_score(bh, bt, visual, hre, him, h, tre, tim, t, rre, rim, r,
                 tm, wp, bp):
    row2 = lambda: pl.BlockSpec((BM, 2 * DIM), lambda i, *_: (i, 0))
    row1 = lambda: pl.BlockSpec((BM, DIM), lambda i, *_: (i, 0))
    grid_spec = pltpu.PrefetchScalarGridSpec(
        num_scalar_prefetch=3,
        grid=(NB,),
        in_specs=[
            pl.BlockSpec(memory_space=pltpu.MemorySpace.HBM),
            row1(), row1(), row2(),
            row1(), row1(), row2(),
            row1(), row1(), row2(),
            pl.BlockSpec((1, 1, BM), lambda i, *_: (i, 0, 0)),
            pl.BlockSpec((2 * DIM, VIS), lambda i, *_: (0, 0)),
            pl.BlockSpec((1, 2 * DIM), lambda i, *_: (0, 0)),
        ],
        out_specs=pl.BlockSpec((1, 1, BM), lambda i, *_: (i, 0, 0)),
        scratch_shapes=[
            pltpu.VMEM((2, BM, VIS), jnp.float32),
            pltpu.VMEM((2, BM, VIS), jnp.float32),
            pltpu.SemaphoreType.DMA((2, 2)),
        ],
    )
    out = pl.pallas_call(
        _score_block,
        grid_spec=grid_spec,
        out_shape=jax.ShapeDtypeStruct((NB, 1, BM), jnp.float32),
    )(bh, bt, tm, visual, hre, him, h, tre, tim, t, rre, rim, r,
      tm.reshape(NB, 1, BM), wp, bp.reshape(1, 2 * DIM))
    return out.reshape(B)


def kernel(batch_h, batch_t, batch_r, task_mode, mode,
           ent_re, ent_im, ent_emb, rel_re, rel_im, rel_emb,
           visual, Wp, bp):
    hre = jnp.take(ent_re, batch_h, axis=0)
    him = jnp.take(ent_im, batch_h, axis=0)
    h = jnp.take(ent_emb, batch_h, axis=0)
    tre = jnp.take(ent_re, batch_t, axis=0)
    tim = jnp.take(ent_im, batch_t, axis=0)
    t = jnp.take(ent_emb, batch_t, axis=0)
    rre = jnp.take(rel_re, batch_r, axis=0)
    rim = jnp.take(rel_im, batch_r, axis=0)
    r = jnp.take(rel_emb, batch_r, axis=0)
    return _fused_score(batch_h * 0, batch_t * 0, visual, hre, him, h, tre, tim, t,
                        rre, rim, r, task_mode, Wp, bp)


# speedup vs baseline: 3.3327x; 3.3327x over previous
import functools

import jax
import jax.numpy as jnp
from jax import lax
from jax.experimental import pallas as pl
from jax.experimental.pallas import tpu as pltpu

B = 16384
DIM = 128
VIS = 4096
BM = 256
NB = B // BM


def _score_block(bh_ref, bt_ref, tms_ref,
                 visual_ref,
                 hre_ref, him_ref, h_ref, tre_ref, tim_ref, t_ref,
                 rre_ref, rim_ref, r_ref, tm_ref, wp_ref, bp_ref,
                 out_ref,
                 xh_buf, xt_buf, sem):
    i = pl.program_id(0)
    nb = pl.num_programs(0)

    def issue(block, slot):
        def body(j, carry):
            row = block * BM + j
            pltpu.make_async_copy(
                visual_ref.at[bh_ref[row]], xh_buf.at[slot, j],
                sem.at[slot, 0]).start()
            pltpu.make_async_copy(
                visual_ref.at[bt_ref[row]], xt_buf.at[slot, j],
                sem.at[slot, 1]).start()
            return carry
        lax.fori_loop(0, BM, body, 0, unroll=8)

    def wait(block, slot):
        del block
        pltpu.make_async_copy(
            visual_ref.at[pl.ds(0, BM)], xh_buf.at[slot],
            sem.at[slot, 0]).wait()
        pltpu.make_async_copy(
            visual_ref.at[pl.ds(0, BM)], xt_buf.at[slot],
            sem.at[slot, 1]).wait()

    @pl.when(i == 0)
    def _():
        issue(0, 0)

    @pl.when(i + 1 < nb)
    def _():
        issue(i + 1, (i + 1) % 2)

    slot = i % 2
    wait(i, slot)

    xh = xh_buf[slot]
    xt = xt_buf[slot]
    wp = wp_ref[...]
    bp = bp_ref[...]
    ha = lax.dot_general(xh, wp, (((1,), (1,)), ((), ())),
                         preferred_element_type=jnp.float32) + bp
    ta = lax.dot_general(xt, wp, (((1,), (1,)), ((), ())),
                         preferred_element_type=jnp.float32) + bp
    hre = hre_ref[...]
    him = him_ref[...]
    tre = tre_ref[...]
    tim = tim_ref[...]
    rre = rre_ref[...]
    rim = rim_ref[...]
    h = h_ref[...]
    t = t_ref[...]
    r = r_ref[...]
    c = -jnp.sum(rre * (hre * tre + him * tim) + rim * (hre * tim - him * tre),
                 axis=-1)
    s_tt = jnp.sum(h * t * r, axis=-1)
    s_it = jnp.sum(ha * t * r, axis=-1)
    s_ti = jnp.sum(h * ta * r, axis=-1)
    s_ii = jnp.sum(ha * ta * r, axis=-1)
    tm = tm_ref[0, ...]
    score = jnp.where(tm == 0, c - s_tt, 0.0)
    score = score + jnp.where(tm == 1, 2.0 * c - s_it - s_ti, 0.0)
    score = score + jnp.where(tm == 2, c - s_ii, 0.0)
    out_ref[0, ...] = score


@jax.jit
def _fused_score(bh, bt, visual, hre, him, h, tre, tim, t, rre, rim, r,
                 tm, wp, bp):
    row2 = lambda: pl.BlockSpec((BM, 2 * DIM), lambda i, *_: (i, 0))
    row1 = lambda: pl.BlockSpec((BM, DIM), lambda i, *_: (i, 0))
    grid_spec = pltpu.PrefetchScalarGridSpec(
        num_scalar_prefetch=3,
        grid=(NB,),
        in_specs=[
            pl.BlockSpec(memory_space=pltpu.MemorySpace.HBM),
            row1(), row1(), row2(),
            row1(), row1(), row2(),
            row1(), row1(), row2(),
            pl.BlockSpec((1, 1, BM), lambda i, *_: (i, 0, 0)),
            pl.BlockSpec((2 * DIM, VIS), lambda i, *_: (0, 0)),
            pl.BlockSpec((1, 2 * DIM), lambda i, *_: (0, 0)),
        ],
        out_specs=pl.BlockSpec((1, 1, BM), lambda i, *_: (i, 0, 0)),
        scratch_shapes=[
            pltpu.VMEM((2, BM, VIS), jnp.float32),
            pltpu.VMEM((2, BM, VIS), jnp.float32),
            pltpu.SemaphoreType.DMA((2, 2)),
        ],
    )
    out = pl.pallas_call(
        _score_block,
        grid_spec=grid_spec,
        out_shape=jax.ShapeDtypeStruct((NB, 1, BM), jnp.float32),
    )(bh, bt, tm, visual, hre, him, h, tre, tim, t, rre, rim, r,
      tm.reshape(NB, 1, BM), wp, bp.reshape(1, 2 * DIM))
    return out.reshape(B)


def kernel(batch_h, batch_t, batch_r, task_mode, mode,
           ent_re, ent_im, ent_emb, rel_re, rel_im, rel_emb,
           visual, Wp, bp):
    hre = jnp.take(ent_re, batch_h, axis=0)
    him = jnp.take(ent_im, batch_h, axis=0)
    h = jnp.take(ent_emb, batch_h, axis=0)
    tre = jnp.take(ent_re, batch_t, axis=0)
    tim = jnp.take(ent_im, batch_t, axis=0)
    t = jnp.take(ent_emb, batch_t, axis=0)
    rre = jnp.take(rel_re, batch_r, axis=0)
    rim = jnp.take(rel_im, batch_r, axis=0)
    r = jnp.take(rel_emb, batch_r, axis=0)
    return _fused_score(batch_h, batch_t, visual, hre, him, h, tre, tim, t,
                        rre, rim, r, task_mode, Wp, bp)


# speedup vs baseline: 4.7536x; 1.4264x over previous
import functools

import jax
import jax.numpy as jnp
from jax import lax
from jax.experimental import pallas as pl
from jax.experimental.pallas import tpu as pltpu
from jax.experimental.pallas import tpu_sc as plsc

B = 16384
DIM = 128
D2 = 2 * DIM
VIS = 4096
BM = 256
NB = B // BM

NW = 32
RPW = B // NW
CH = 64
NCH = RPW // CH

_sc_mesh = plsc.VectorSubcoreMesh(core_axis_name="c", subcore_axis_name="s")


@functools.partial(
    pl.kernel,
    mesh=_sc_mesh,
    out_type=[
        jax.ShapeDtypeStruct((B, 16), jnp.float32),
        jax.ShapeDtypeStruct((B, 16), jnp.float32),
        jax.ShapeDtypeStruct((B, D2), jnp.float32),
        jax.ShapeDtypeStruct((B, D2), jnp.float32),
        jax.ShapeDtypeStruct((B, D2), jnp.float32),
    ],
    scratch_types=[
        pltpu.VMEM((RPW,), jnp.int32),
        pltpu.VMEM((RPW,), jnp.int32),
        pltpu.VMEM((RPW,), jnp.int32),
        pltpu.VMEM((CH, DIM), jnp.float32),
        pltpu.VMEM((CH, DIM), jnp.float32),
        pltpu.VMEM((CH, DIM), jnp.float32),
        pltpu.VMEM((CH, DIM), jnp.float32),
        pltpu.VMEM((CH, DIM), jnp.float32),
        pltpu.VMEM((CH, DIM), jnp.float32),
        pltpu.VMEM((CH, D2), jnp.float32),
        pltpu.VMEM((CH, D2), jnp.float32),
        pltpu.VMEM((CH, D2), jnp.float32),
        pltpu.VMEM((CH, 16), jnp.float32),
        pltpu.VMEM((CH, 16), jnp.float32),
        pltpu.SemaphoreType.DMA,
    ],
)
def _sc_prep(bh_hbm, bt_hbm, br_hbm,
             entre_hbm, entim_hbm, entemb_hbm,
             relre_hbm, relim_hbm, relemb_hbm,
             c_hbm, stt_hbm, u_hbm, w_hbm, r_hbm,
             bh_v, bt_v, br_v,
             hre_v, him_v, tre_v, tim_v, rre_v, rim_v,
             h_v, t_v, r_v, c_v, stt_v, sem):
    wid = lax.axis_index("s") * 2 + lax.axis_index("c")
    rbase = wid * RPW
    pltpu.sync_copy(bh_hbm.at[pl.ds(rbase, RPW)], bh_v)
    pltpu.sync_copy(bt_hbm.at[pl.ds(rbase, RPW)], bt_v)
    pltpu.sync_copy(br_hbm.at[pl.ds(rbase, RPW)], br_v)

    def chunk_body(ci, carry):
        off = ci * CH
        ih = bh_v.at[pl.ds(off, CH)]
        it = bt_v.at[pl.ds(off, CH)]
        ir = br_v.at[pl.ds(off, CH)]
        cps = [
            pltpu.make_async_copy(entre_hbm.at[ih], hre_v, sem),
            pltpu.make_async_copy(entim_hbm.at[ih], him_v, sem),
            pltpu.make_async_copy(entre_hbm.at[it], tre_v, sem),
            pltpu.make_async_copy(entim_hbm.at[it], tim_v, sem),
            pltpu.make_async_copy(relre_hbm.at[ir], rre_v, sem),
            pltpu.make_async_copy(relim_hbm.at[ir], rim_v, sem),
            pltpu.make_async_copy(entemb_hbm.at[ih], h_v, sem),
            pltpu.make_async_copy(entemb_hbm.at[it], t_v, sem),
            pltpu.make_async_copy(relemb_hbm.at[ir], r_v, sem),
        ]
        for cp in cps:
            cp.start()
        for cp in cps:
            cp.wait()

        def row_body(row, carry2):
                def ck(k, acc):
                    sl = pl.ds(k * 16, 16)
                    hre = hre_v[row, sl]
                    him = him_v[row, sl]
                    tre = tre_v[row, sl]
                    tim = tim_v[row, sl]
                    rre = rre_v[row, sl]
                    rim = rim_v[row, sl]
                    return acc + (rre * (hre * tre + him * tim)
                                  + rim * (hre * tim - him * tre))

                accc = lax.fori_loop(0, DIM // 16, ck,
                                     jnp.zeros((16,), jnp.float32), unroll=8)

                def ck2(k, acc):
                    sl = pl.ds(k * 16, 16)
                    hh = h_v[row, sl]
                    tt = t_v[row, sl]
                    rr = r_v[row, sl]
                    trr = tt * rr
                    t_v[row, sl] = trr
                    h_v[row, sl] = hh * rr
                    return acc + hh * trr

                accs = lax.fori_loop(0, D2 // 16, ck2,
                                     jnp.zeros((16,), jnp.float32), unroll=8)
                c_v[row] = accc
                stt_v[row] = accs
                return carry2

        lax.fori_loop(0, CH, row_body, 0)
        pltpu.sync_copy(t_v, u_hbm.at[pl.ds(rbase + off, CH)])
        pltpu.sync_copy(h_v, w_hbm.at[pl.ds(rbase + off, CH)])
        pltpu.sync_copy(r_v, r_hbm.at[pl.ds(rbase + off, CH)])
        pltpu.sync_copy(c_v, c_hbm.at[pl.ds(rbase + off, CH)])
        pltpu.sync_copy(stt_v, stt_hbm.at[pl.ds(rbase + off, CH)])
        return carry

    lax.fori_loop(0, NCH, chunk_body, 0)


def _score_block(bh_ref, bt_ref,
                 visual_ref,
                 u_ref, w_ref, rm_ref, c_ref, stt_ref, tm_ref, wp_ref, bp_ref,
                 out_ref,
                 xh_buf, xt_buf, sem):
    i = pl.program_id(0)
    nb = pl.num_programs(0)

    def issue(block, slot):
        def body(j, carry):
            row = block * BM + j
            pltpu.make_async_copy(
                visual_ref.at[bh_ref[row]], xh_buf.at[slot, j],
                sem.at[slot, 0]).start()
            pltpu.make_async_copy(
                visual_ref.at[bt_ref[row]], xt_buf.at[slot, j],
                sem.at[slot, 1]).start()
            return carry
        lax.fori_loop(0, BM, body, 0, unroll=8)

    def wait(slot):
        pltpu.make_async_copy(
            visual_ref.at[pl.ds(0, BM)], xh_buf.at[slot],
            sem.at[slot, 0]).wait()
        pltpu.make_async_copy(
            visual_ref.at[pl.ds(0, BM)], xt_buf.at[slot],
            sem.at[slot, 1]).wait()

    @pl.when(i == 0)
    def _():
        issue(0, 0)

    @pl.when(i + 1 < nb)
    def _():
        issue(i + 1, (i + 1) % 2)

    slot = i % 2
    wait(slot)

    xh = xh_buf[slot]
    xt = xt_buf[slot]
    wp = wp_ref[...]
    bp = bp_ref[...]
    ha = lax.dot_general(xh, wp, (((1,), (1,)), ((), ())),
                         preferred_element_type=jnp.float32) + bp
    ta = lax.dot_general(xt, wp, (((1,), (1,)), ((), ())),
                         preferred_element_type=jnp.float32) + bp
    u = u_ref[...]
    w = w_ref[...]
    rm = rm_ref[...]
    c = -jnp.sum(c_ref[0], axis=-1)
    stt = jnp.sum(stt_ref[0], axis=-1)
    tm = tm_ref[0, ...]
    s_hyb = jnp.sum(ha * u + ta * w, axis=-1)
    s_ii = jnp.sum(ha * ta * rm, axis=-1)
    score = jnp.where(tm == 0, c - stt, 0.0)
    score = score + jnp.where(tm == 1, 2.0 * c - s_hyb, 0.0)
    score = score + jnp.where(tm == 2, c - s_ii, 0.0)
    out_ref[0, ...] = score


@jax.jit
def _fused_score(bh, bt, visual, u, w, rm, c, stt, tm, wp, bp):
    row2 = lambda: pl.BlockSpec((BM, D2), lambda i, *_: (i, 0))
    vec1 = lambda: pl.BlockSpec((1, 1, BM), lambda i, *_: (i, 0, 0))
    grid_spec = pltpu.PrefetchScalarGridSpec(
        num_scalar_prefetch=2,
        grid=(NB,),
        in_specs=[
            pl.BlockSpec(memory_space=pltpu.MemorySpace.HBM),
            row2(), row2(), row2(),
            pl.BlockSpec((1, BM, 16), lambda i, *_: (i, 0, 0)),
            pl.BlockSpec((1, BM, 16), lambda i, *_: (i, 0, 0)),
            vec1(),
            pl.BlockSpec((D2, VIS), lambda i, *_: (0, 0)),
            pl.BlockSpec((1, D2), lambda i, *_: (0, 0)),
        ],
        out_specs=pl.BlockSpec((1, 1, BM), lambda i, *_: (i, 0, 0)),
        scratch_shapes=[
            pltpu.VMEM((2, BM, VIS), jnp.float32),
            pltpu.VMEM((2, BM, VIS), jnp.float32),
            pltpu.SemaphoreType.DMA((2, 2)),
        ],
    )
    out = pl.pallas_call(
        _score_block,
        grid_spec=grid_spec,
        out_shape=jax.ShapeDtypeStruct((NB, 1, BM), jnp.float32),
    )(bh, bt, visual, u, w, rm,
      c.reshape(NB, BM, 16), stt.reshape(NB, BM, 16), tm.reshape(NB, 1, BM),
      wp, bp.reshape(1, D2))
    return out.reshape(B)


def kernel(batch_h, batch_t, batch_r, task_mode, mode,
           ent_re, ent_im, ent_emb, rel_re, rel_im, rel_emb,
           visual, Wp, bp):
    c, stt, u, w, r = _sc_prep(batch_h, batch_t, batch_r,
                               ent_re, ent_im, ent_emb,
                               rel_re, rel_im, rel_emb)
    return _fused_score(batch_h, batch_t, visual, u, w, r, c, stt,
                        task_mode, Wp, bp)
